# speedup vs baseline: 1.0433x; 1.0433x over previous
import functools

import jax
import jax.numpy as jnp
from jax import lax
from jax.experimental import pallas as pl
from jax.experimental.pallas import tpu as pltpu

N = 10000
D = 128
BM = 2000


def _mm_body(a_ref, w_ref, b_ref, o_ref, *, act):
    y = jnp.dot(a_ref[...], w_ref[...], preferred_element_type=jnp.float32)
    y = y + b_ref[...]
    if act == "relu":
        y = jnp.maximum(y, 0.0)
    o_ref[...] = y


def _mm(a, w, b, act="none"):
    return pl.pallas_call(
        functools.partial(_mm_body, act=act),
        grid=(N // BM,),
        in_specs=[
            pl.BlockSpec((BM, D), lambda i: (i, 0)),
            pl.BlockSpec((D, D), lambda i: (0, 0)),
            pl.BlockSpec((1, D), lambda i: (0, 0)),
        ],
        out_specs=pl.BlockSpec((BM, D), lambda i: (i, 0)),
        out_shape=jax.ShapeDtypeStruct((N, D), jnp.float32),
    )(a, w, b.reshape(1, D))


def _masked_mm_body(x_ref, m_ref, w_ref, b_ref, o_ref):
    xm = x_ref[...] * m_ref[...]
    o_ref[...] = jnp.dot(xm, w_ref[...], preferred_element_type=jnp.float32) + b_ref[...]


def _masked_mm(x, mask, w, b):
    return pl.pallas_call(
        _masked_mm_body,
        grid=(N // BM,),
        in_specs=[
            pl.BlockSpec((BM, D), lambda i: (i, 0)),
            pl.BlockSpec((BM, D), lambda i: (i, 0)),
            pl.BlockSpec((D, D), lambda i: (0, 0)),
            pl.BlockSpec((1, D), lambda i: (0, 0)),
        ],
        out_specs=pl.BlockSpec((BM, D), lambda i: (i, 0)),
        out_shape=jax.ShapeDtypeStruct((N, D), jnp.float32),
    )(x, mask, w, b.reshape(1, D))


def _mask_mlp_body(ag_ref, xl2_ref, m1t_ref, m1b_ref, b1_ref, m2_ref, b2_ref, o_ref):
    a = jnp.maximum(ag_ref[...], 0.0)
    c = jnp.maximum(xl2_ref[...], 0.0)
    w = (jnp.dot(a, m1t_ref[...], preferred_element_type=jnp.float32)
         + jnp.dot(c, m1b_ref[...], preferred_element_type=jnp.float32)
         + b1_ref[...])
    w = jnp.maximum(w, 0.0)
    y = jnp.dot(w, m2_ref[...], preferred_element_type=jnp.float32) + b2_ref[...]
    o_ref[...] = jax.nn.sigmoid(y)


def _mask_mlp(aggr, xl2, m1W, m1b, m2W, m2b):
    return pl.pallas_call(
        _mask_mlp_body,
        grid=(N // BM,),
        in_specs=[
            pl.BlockSpec((BM, D), lambda i: (i, 0)),
            pl.BlockSpec((BM, D), lambda i: (i, 0)),
            pl.BlockSpec((D, D), lambda i: (0, 0)),
            pl.BlockSpec((D, D), lambda i: (0, 0)),
            pl.BlockSpec((1, D), lambda i: (0, 0)),
            pl.BlockSpec((D, D), lambda i: (0, 0)),
            pl.BlockSpec((1, D), lambda i: (0, 0)),
        ],
        out_specs=pl.BlockSpec((BM, D), lambda i: (i, 0)),
        out_shape=jax.ShapeDtypeStruct((N, D), jnp.float32),
    )(aggr, xl2, m1W[:D], m1W[D:], m1b.reshape(1, D), m2W, m2b.reshape(1, D))


def _combine_body(ag_ref, x_ref, lw_ref, m_ref, o_ref):
    y = ag_ref[...] + jnp.dot(x_ref[...], lw_ref[...], preferred_element_type=jnp.float32)
    o_ref[...] = jnp.maximum(y * m_ref[...], 0.0)


def _combine(aggr, x, linW, mask):
    return pl.pallas_call(
        _combine_body,
        grid=(N // BM,),
        in_specs=[
            pl.BlockSpec((BM, D), lambda i: (i, 0)),
            pl.BlockSpec((BM, D), lambda i: (i, 0)),
            pl.BlockSpec((D, D), lambda i: (0, 0)),
            pl.BlockSpec((BM, D), lambda i: (i, 0)),
        ],
        out_specs=pl.BlockSpec((BM, D), lambda i: (i, 0)),
        out_shape=jax.ShapeDtypeStruct((N, D), jnp.float32),
    )(aggr, x, linW, mask)


def _post_body(x_ref, p1_ref, b1_ref, p2_ref, b2_ref, o_ref):
    y = jnp.dot(x_ref[...], p1_ref[...], preferred_element_type=jnp.float32) + b1_ref[...]
    y = jnp.maximum(y, 0.0)
    o_ref[...] = jnp.dot(y, p2_ref[...], preferred_element_type=jnp.float32) + b2_ref[...]


def _post(x, p1W, p1b, p2W, p2b):
    return pl.pallas_call(
        _post_body,
        grid=(N // BM,),
        in_specs=[
            pl.BlockSpec((BM, D), lambda i: (i, 0)),
            pl.BlockSpec((D, D), lambda i: (0, 0)),
            pl.BlockSpec((1, D), lambda i: (0, 0)),
            pl.BlockSpec((D, D), lambda i: (0, 0)),
            pl.BlockSpec((1, D), lambda i: (0, 0)),
        ],
        out_specs=pl.BlockSpec((BM, D), lambda i: (i, 0)),
        out_shape=jax.ShapeDtypeStruct((N, D), jnp.float32),
    )(x, p1W, p1b.reshape(1, D), p2W, p2b.reshape(1, D))


def _segsum(h, src, dst, ew):
    msg = ew[:, None] * h[src]
    return jax.ops.segment_sum(msg, dst, num_segments=N)


def kernel(x, edge_attr, edge_index, W0, lin0W, W1, lin1W,
           m0_l1W, m0_l1b, m0_l2W, m0_l2b, m0_m1W, m0_m1b, m0_m2W, m0_m2b,
           m1_l1W, m1_l1b, m1_l2W, m1_l2b, m1_m1W, m1_m1b, m1_m2W, m1_m2b,
           p1W, p1b, p2W, p2b):
    src = edge_index[0]
    dst = edge_index[1]
    ew = edge_attr

    h1 = _mm(x, m0_l1W, m0_l1b)
    aggr1 = _segsum(h1, src, dst, ew)
    xl2 = _mm(x, m0_l2W, m0_l2b)
    mask0 = _mask_mlp(aggr1, xl2, m0_m1W, m0_m1b, m0_m2W, m0_m2b)

    h2 = _masked_mm(x, mask0, W0, jnp.zeros((D,), jnp.float32))
    aggr2 = _segsum(h2, src, dst, ew)
    x1 = _combine(aggr2, x, lin0W, mask0)

    h1 = _masked_mm(x1, mask0, m1_l1W, m1_l1b)
    aggr1 = _segsum(h1, src, dst, ew)
    xl2 = _masked_mm(x1, mask0, m1_l2W, m1_l2b)
    mask1 = _mask_mlp(aggr1, xl2, m1_m1W, m1_m1b, m1_m2W, m1_m2b)

    h2 = _masked_mm(x1, mask1, W1, jnp.zeros((D,), jnp.float32))
    aggr2 = _segsum(h2, src, dst, ew)
    x2 = _combine(aggr2, x1, lin1W, mask1)

    return _post(x2, p1W, p1b, p2W, p2b)


# speedup vs baseline: 2.3190x; 2.2227x over previous
import functools

import jax
import jax.numpy as jnp
from jax import lax
from jax.experimental import pallas as pl
from jax.experimental.pallas import tpu as pltpu

N = 10000
D = 128
BM = 2000


def _mm_body(a_ref, w_ref, b_ref, o_ref, *, act):
    y = jnp.dot(a_ref[...], w_ref[...], preferred_element_type=jnp.float32)
    y = y + b_ref[...]
    if act == "relu":
        y = jnp.maximum(y, 0.0)
    o_ref[...] = y


def _mm(a, w, b, act="none"):
    return pl.pallas_call(
        functools.partial(_mm_body, act=act),
        grid=(N // BM,),
        in_specs=[
            pl.BlockSpec((BM, D), lambda i: (i, 0)),
            pl.BlockSpec((D, D), lambda i: (0, 0)),
            pl.BlockSpec((1, D), lambda i: (0, 0)),
        ],
        out_specs=pl.BlockSpec((BM, D), lambda i: (i, 0)),
        out_shape=jax.ShapeDtypeStruct((N, D), jnp.float32),
    )(a, w, b.reshape(1, D))


def _masked_mm_body(x_ref, m_ref, w_ref, b_ref, o_ref):
    xm = x_ref[...] * m_ref[...]
    o_ref[...] = jnp.dot(xm, w_ref[...], preferred_element_type=jnp.float32) + b_ref[...]


def _masked_mm(x, mask, w, b):
    return pl.pallas_call(
        _masked_mm_body,
        grid=(N // BM,),
        in_specs=[
            pl.BlockSpec((BM, D), lambda i: (i, 0)),
            pl.BlockSpec((BM, D), lambda i: (i, 0)),
            pl.BlockSpec((D, D), lambda i: (0, 0)),
            pl.BlockSpec((1, D), lambda i: (0, 0)),
        ],
        out_specs=pl.BlockSpec((BM, D), lambda i: (i, 0)),
        out_shape=jax.ShapeDtypeStruct((N, D), jnp.float32),
    )(x, mask, w, b.reshape(1, D))


def _mask_mlp_body(agA_ref, agB_ref, xl2_ref, m1t_ref, m1b_ref, b1_ref, m2_ref, b2_ref, o_ref):
    a = jnp.maximum(agA_ref[...] + agB_ref[...], 0.0)
    c = jnp.maximum(xl2_ref[...], 0.0)
    w = (jnp.dot(a, m1t_ref[...], preferred_element_type=jnp.float32)
         + jnp.dot(c, m1b_ref[...], preferred_element_type=jnp.float32)
         + b1_ref[...])
    w = jnp.maximum(w, 0.0)
    y = jnp.dot(w, m2_ref[...], preferred_element_type=jnp.float32) + b2_ref[...]
    o_ref[...] = jax.nn.sigmoid(y)


def _mask_mlp(agg2, xl2, m1W, m1b, m2W, m2b):
    return pl.pallas_call(
        _mask_mlp_body,
        grid=(N // BM,),
        in_specs=[
            pl.BlockSpec((BM, D), lambda i: (i, 0)),
            pl.BlockSpec((BM, D), lambda i: (i, 0)),
            pl.BlockSpec((BM, D), lambda i: (i, 0)),
            pl.BlockSpec((D, D), lambda i: (0, 0)),
            pl.BlockSpec((D, D), lambda i: (0, 0)),
            pl.BlockSpec((1, D), lambda i: (0, 0)),
            pl.BlockSpec((D, D), lambda i: (0, 0)),
            pl.BlockSpec((1, D), lambda i: (0, 0)),
        ],
        out_specs=pl.BlockSpec((BM, D), lambda i: (i, 0)),
        out_shape=jax.ShapeDtypeStruct((N, D), jnp.float32),
    )(agg2[:N], agg2[N_PAD:N_PAD + N], xl2, m1W[:D], m1W[D:], m1b.reshape(1, D), m2W,
      m2b.reshape(1, D))


def _combine_body(agA_ref, agB_ref, x_ref, lw_ref, m_ref, o_ref):
    y = (agA_ref[...] + agB_ref[...]
         + jnp.dot(x_ref[...], lw_ref[...], preferred_element_type=jnp.float32))
    o_ref[...] = jnp.maximum(y * m_ref[...], 0.0)


def _combine(agg2, x, linW, mask):
    return pl.pallas_call(
        _combine_body,
        grid=(N // BM,),
        in_specs=[
            pl.BlockSpec((BM, D), lambda i: (i, 0)),
            pl.BlockSpec((BM, D), lambda i: (i, 0)),
            pl.BlockSpec((BM, D), lambda i: (i, 0)),
            pl.BlockSpec((D, D), lambda i: (0, 0)),
            pl.BlockSpec((BM, D), lambda i: (i, 0)),
        ],
        out_specs=pl.BlockSpec((BM, D), lambda i: (i, 0)),
        out_shape=jax.ShapeDtypeStruct((N, D), jnp.float32),
    )(agg2[:N], agg2[N_PAD:N_PAD + N], x, linW, mask)


def _post_body(x_ref, p1_ref, b1_ref, p2_ref, b2_ref, o_ref):
    y = jnp.dot(x_ref[...], p1_ref[...], preferred_element_type=jnp.float32) + b1_ref[...]
    y = jnp.maximum(y, 0.0)
    o_ref[...] = jnp.dot(y, p2_ref[...], preferred_element_type=jnp.float32) + b2_ref[...]


def _post(x, p1W, p1b, p2W, p2b):
    return pl.pallas_call(
        _post_body,
        grid=(N // BM,),
        in_specs=[
            pl.BlockSpec((BM, D), lambda i: (i, 0)),
            pl.BlockSpec((D, D), lambda i: (0, 0)),
            pl.BlockSpec((1, D), lambda i: (0, 0)),
            pl.BlockSpec((D, D), lambda i: (0, 0)),
            pl.BlockSpec((1, D), lambda i: (0, 0)),
        ],
        out_specs=pl.BlockSpec((BM, D), lambda i: (i, 0)),
        out_shape=jax.ShapeDtypeStruct((N, D), jnp.float32),
    )(x, p1W, p1b.reshape(1, D), p2W, p2b.reshape(1, D))


NC = 2
NS = 16
CB = 128
E_PAD = 327680
EPT = E_PAD // (NC * NS)
N_PAD = 10240
_sc_mesh = None


def _get_sc_mesh():
    global _sc_mesh
    if _sc_mesh is None:
        from jax.experimental.pallas import tpu_sc as plsc
        _sc_mesh = plsc.VectorSubcoreMesh(
            core_axis_name="c", subcore_axis_name="s",
            num_cores=NC, num_subcores=NS)
    return _sc_mesh


def _segsum_sc_body(h_hbm, src_hbm, dst_hbm, ew_hbm, out_hbm,
                    src_v, dst_v, ew_v, rows_v, acc_sh, sem):
    from jax.experimental.pallas import tpu_sc as plsc
    c = lax.axis_index("c")
    s = lax.axis_index("s")

    z = jnp.zeros((16,), jnp.float32)

    def zrow(r, _):
        for f in range(D // 16):
            rows_v[r, pl.ds(f * 16, 16)] = z
        return 0

    lax.fori_loop(0, CB, zrow, 0, unroll=4)
    rpt = N_PAD // NS
    row0 = s * rpt
    off = 0
    while off < rpt:
        nr = min(CB, rpt - off)
        pltpu.sync_copy(rows_v.at[pl.ds(0, nr)], acc_sh.at[pl.ds(row0 + off, nr)])
        off += nr
    plsc.subcore_barrier()

    base_t = (s * NC + c) * EPT

    def chunk(i, _):
        base = base_t + i * CB
        pltpu.sync_copy(src_hbm.at[pl.ds(base, CB)], src_v)
        pltpu.sync_copy(ew_hbm.at[pl.ds(base, CB)], ew_v)
        pltpu.async_copy(h_hbm.at[src_v], rows_v, sem).wait()

        def sgroup(g, _):
            base_r = g * 16
            ewg = ew_v[pl.ds(g * 16, 16)]
            for r2 in range(16):
                m = jnp.broadcast_to(ewg[r2], (16,))
                for f in range(D // 16):
                    sl = pl.ds(f * 16, 16)
                    rows_v[base_r + r2, sl] = rows_v[base_r + r2, sl] * m
            return 0

        lax.fori_loop(0, CB // 16, sgroup, 0)
        pltpu.sync_copy(dst_hbm.at[pl.ds(base, CB)], dst_v)
        pltpu.sync_copy(rows_v, acc_sh.at[dst_v], add=True)
        return 0

    lax.fori_loop(0, EPT // CB, chunk, 0)

    plsc.subcore_barrier()
    off = 0
    while off < rpt:
        nr = min(CB, rpt - off)
        pltpu.sync_copy(acc_sh.at[pl.ds(row0 + off, nr)],
                        out_hbm.at[pl.ds(c * N_PAD + row0 + off, nr)])
        off += nr


def _segsum_partials(h, src_p, dst_p, ew_p):
    f = pl.kernel(
        _segsum_sc_body,
        out_type=jax.ShapeDtypeStruct((NC * N_PAD, D), jnp.float32),
        mesh=_get_sc_mesh(),
        scratch_types=[
            pltpu.VMEM((CB,), jnp.int32),
            pltpu.VMEM((CB,), jnp.int32),
            pltpu.VMEM((CB,), jnp.float32),
            pltpu.VMEM((CB, D), jnp.float32),
            pltpu.VMEM_SHARED((N_PAD, D), jnp.float32),
            pltpu.SemaphoreType.DMA,
        ],
    )
    return f(h, src_p, dst_p, ew_p)


def kernel(x, edge_attr, edge_index, W0, lin0W, W1, lin1W,
           m0_l1W, m0_l1b, m0_l2W, m0_l2b, m0_m1W, m0_m1b, m0_m2W, m0_m2b,
           m1_l1W, m1_l1b, m1_l2W, m1_l2b, m1_m1W, m1_m1b, m1_m2W, m1_m2b,
           p1W, p1b, p2W, p2b):
    pad = E_PAD - edge_attr.shape[0]
    src = jnp.pad(edge_index[0], (0, pad))
    dst = jnp.pad(edge_index[1], (0, pad))
    ew = jnp.pad(edge_attr, (0, pad))

    h1 = _mm(x, m0_l1W, m0_l1b)
    aggr1 = _segsum_partials(h1, src, dst, ew)
    xl2 = _mm(x, m0_l2W, m0_l2b)
    mask0 = _mask_mlp(aggr1, xl2, m0_m1W, m0_m1b, m0_m2W, m0_m2b)

    h2 = _masked_mm(x, mask0, W0, jnp.zeros((D,), jnp.float32))
    aggr2 = _segsum_partials(h2, src, dst, ew)
    x1 = _combine(aggr2, x, lin0W, mask0)

    h1 = _masked_mm(x1, mask0, m1_l1W, m1_l1b)
    aggr1 = _segsum_partials(h1, src, dst, ew)
    xl2 = _masked_mm(x1, mask0, m1_l2W, m1_l2b)
    mask1 = _mask_mlp(aggr1, xl2, m1_m1W, m1_m1b, m1_m2W, m1_m2b)

    h2 = _masked_mm(x1, mask1, W1, jnp.zeros((D,), jnp.float32))
    aggr2 = _segsum_partials(h2, src, dst, ew)
    x2 = _combine(aggr2, x1, lin1W, mask1)

    return _post(x2, p1W, p1b, p2W, p2b)


# speedup vs baseline: 3.2083x; 1.3835x over previous
import functools

import jax
import jax.numpy as jnp
from jax import lax
from jax.experimental import pallas as pl
from jax.experimental.pallas import tpu as pltpu

N = 10000
D = 128
BM = 2000

NC = 2
NS = 16
NT = NC * NS
CB = 128
E_PAD = 327680
EPT = E_PAD // NT
NCH = EPT // CB
RPT = 632
NR = NS * RPT
RD = 4

_sc_mesh = None


def _get_sc_mesh():
    global _sc_mesh
    if _sc_mesh is None:
        from jax.experimental.pallas import tpu_sc as plsc
        _sc_mesh = plsc.VectorSubcoreMesh(
            core_axis_name="c", subcore_axis_name="s",
            num_cores=NC, num_subcores=NS)
    return _sc_mesh


def _mm_body(a_ref, w_ref, b_ref, o_ref):
    y = jnp.dot(a_ref[...], w_ref[...], preferred_element_type=jnp.float32)
    o_ref[...] = y + b_ref[...]


def _mm(a, w, b):
    return pl.pallas_call(
        _mm_body,
        grid=(N // BM,),
        in_specs=[
            pl.BlockSpec((BM, D), lambda i: (i, 0)),
            pl.BlockSpec((D, D), lambda i: (0, 0)),
            pl.BlockSpec((1, D), lambda i: (0, 0)),
        ],
        out_specs=pl.BlockSpec((BM, D), lambda i: (i, 0)),
        out_shape=jax.ShapeDtypeStruct((N, D), jnp.float32),
    )(a, w, b.reshape(1, D))


def _masked_mm_body(x_ref, m_ref, w_ref, b_ref, o_ref):
    xm = x_ref[...] * m_ref[...]
    o_ref[...] = jnp.dot(xm, w_ref[...], preferred_element_type=jnp.float32) + b_ref[...]


def _masked_mm(x, mask, w, b):
    return pl.pallas_call(
        _masked_mm_body,
        grid=(N // BM,),
        in_specs=[
            pl.BlockSpec((BM, D), lambda i: (i, 0)),
            pl.BlockSpec((BM, D), lambda i: (i, 0)),
            pl.BlockSpec((D, D), lambda i: (0, 0)),
            pl.BlockSpec((1, D), lambda i: (0, 0)),
        ],
        out_specs=pl.BlockSpec((BM, D), lambda i: (i, 0)),
        out_shape=jax.ShapeDtypeStruct((N, D), jnp.float32),
    )(x, mask, w, b.reshape(1, D))


def _mask_mlp_body(agA_ref, agB_ref, xl2_ref, m1t_ref, m1b_ref, b1_ref,
                   m2_ref, b2_ref, o_ref):
    a = jnp.maximum(agA_ref[...] + agB_ref[...], 0.0)
    cx = jnp.maximum(xl2_ref[...], 0.0)
    w = (jnp.dot(a, m1t_ref[...], preferred_element_type=jnp.float32)
         + jnp.dot(cx, m1b_ref[...], preferred_element_type=jnp.float32)
         + b1_ref[...])
    w = jnp.maximum(w, 0.0)
    y = jnp.dot(w, m2_ref[...], preferred_element_type=jnp.float32) + b2_ref[...]
    o_ref[...] = jax.nn.sigmoid(y)


def _mask_mlp(agA, agB, xl2, m1W, m1b, m2W, m2b):
    return pl.pallas_call(
        _mask_mlp_body,
        grid=(N // BM,),
        in_specs=[
            pl.BlockSpec((BM, D), lambda i: (i, 0)),
            pl.BlockSpec((BM, D), lambda i: (i, 0)),
            pl.BlockSpec((BM, D), lambda i: (i, 0)),
            pl.BlockSpec((D, D), lambda i: (0, 0)),
            pl.BlockSpec((D, D), lambda i: (0, 0)),
            pl.BlockSpec((1, D), lambda i: (0, 0)),
            pl.BlockSpec((D, D), lambda i: (0, 0)),
            pl.BlockSpec((1, D), lambda i: (0, 0)),
        ],
        out_specs=pl.BlockSpec((BM, D), lambda i: (i, 0)),
        out_shape=jax.ShapeDtypeStruct((N, D), jnp.float32),
    )(agA, agB, xl2, m1W[:D], m1W[D:], m1b.reshape(1, D), m2W,
      m2b.reshape(1, D))


def _combine_body(agA_ref, agB_ref, x_ref, lw_ref, m_ref, o_ref):
    y = (agA_ref[...] + agB_ref[...]
         + jnp.dot(x_ref[...], lw_ref[...], preferred_element_type=jnp.float32))
    o_ref[...] = jnp.maximum(y * m_ref[...], 0.0)


def _combine(agA, agB, x, linW, mask):
    return pl.pallas_call(
        _combine_body,
        grid=(N // BM,),
        in_specs=[
            pl.BlockSpec((BM, D), lambda i: (i, 0)),
            pl.BlockSpec((BM, D), lambda i: (i, 0)),
            pl.BlockSpec((BM, D), lambda i: (i, 0)),
            pl.BlockSpec((D, D), lambda i: (0, 0)),
            pl.BlockSpec((BM, D), lambda i: (i, 0)),
        ],
        out_specs=pl.BlockSpec((BM, D), lambda i: (i, 0)),
        out_shape=jax.ShapeDtypeStruct((N, D), jnp.float32),
    )(agA, agB, x, linW, mask)


def _post_body(x_ref, p1_ref, b1_ref, p2_ref, b2_ref, o_ref):
    y = jnp.dot(x_ref[...], p1_ref[...], preferred_element_type=jnp.float32) + b1_ref[...]
    y = jnp.maximum(y, 0.0)
    o_ref[...] = jnp.dot(y, p2_ref[...], preferred_element_type=jnp.float32) + b2_ref[...]


def _post(x, p1W, p1b, p2W, p2b):
    return pl.pallas_call(
        _post_body,
        grid=(N // BM,),
        in_specs=[
            pl.BlockSpec((BM, D), lambda i: (i, 0)),
            pl.BlockSpec((D, D), lambda i: (0, 0)),
            pl.BlockSpec((1, D), lambda i: (0, 0)),
            pl.BlockSpec((D, D), lambda i: (0, 0)),
            pl.BlockSpec((1, D), lambda i: (0, 0)),
        ],
        out_specs=pl.BlockSpec((BM, D), lambda i: (i, 0)),
        out_shape=jax.ShapeDtypeStruct((N, D), jnp.float32),
    )(x, p1W, p1b.reshape(1, D), p2W, p2b.reshape(1, D))


def _segsum_sc_body(h_hbm, src_hbm, dst_hbm, ew_hbm, out_hbm,
                    sring, dring, ew_v, rA, rB, acc_sh,
                    isS0, isS1, isS2, isS3, isD0, isD1, isD2, isD3,
                    g0, g1, s0, s1):
    from jax.experimental.pallas import tpu_sc as plsc
    c = lax.axis_index("c")
    s = lax.axis_index("s")
    wid = s * NC + c
    ch0 = wid * NCH

    isS = (isS0, isS1, isS2, isS3)
    isD = (isD0, isD1, isD2, isD3)
    gse = (g0, g1)
    sse = (s0, s1)
    bufs = (rA, rB)

    pltpu.sync_copy(ew_hbm.at[pl.ds(wid * EPT, EPT)], ew_v)

    z = jnp.zeros((16,), jnp.float32)

    def zrow(r, _):
        for f in range(D // 16):
            rA[r, pl.ds(f * 16, 16)] = z
        return 0

    lax.fori_loop(0, CB, zrow, 0, unroll=4)
    row0 = s * RPT
    for off in range(0, RPT, CB):
        nr = min(CB, RPT - off)
        pltpu.sync_copy(rA.at[pl.ds(0, nr)], acc_sh.at[pl.ds(row0 + off, nr)])
    plsc.subcore_barrier()

    def scale(buf, ch):
        def sgroup(g, _):
            ewg = ew_v[pl.ds(ch * CB + g * 16, 16)]
            for r2 in range(16):
                m = jnp.broadcast_to(ewg[r2], (16,))
                for f in range(D // 16):
                    sl = pl.ds(f * 16, 16)
                    buf[g * 16 + r2, sl] = buf[g * 16 + r2, sl] * m
            return 0

        lax.fori_loop(0, CB // 16, sgroup, 0)

    def issue_idx(j, slot):
        pltpu.async_copy(src_hbm.at[ch0 + j], sring.at[slot], isS[slot])
        pltpu.async_copy(dst_hbm.at[ch0 + j], dring.at[slot], isD[slot])

    def wait_idxS(j, slot):
        pltpu.make_async_copy(src_hbm.at[ch0 + j], sring.at[slot],
                              isS[slot]).wait()

    def wait_idxD(j, slot):
        pltpu.make_async_copy(dst_hbm.at[ch0 + j], dring.at[slot],
                              isD[slot]).wait()

    def issue_gather(j, slot, b):
        pltpu.async_copy(h_hbm.at[sring.at[slot]], bufs[b], gse[b])

    def wait_gather(j, slot, b):
        pltpu.make_async_copy(h_hbm.at[sring.at[slot]], bufs[b],
                              gse[b]).wait()

    def issue_scatter(j, slot, b):
        pltpu.async_copy(bufs[b], acc_sh.at[dring.at[slot]], sse[b], add=True)

    def wait_scatter(j, slot, b):
        pltpu.make_async_copy(bufs[b], acc_sh.at[dring.at[slot]],
                              sse[b]).wait()

    def step(j, k, first, has_next, has_idx3):
        if has_next:
            wait_idxS(j + 1, (k + 1) % RD)
        if not first:
            wait_scatter(j - 1, (k + RD - 1) % RD, (k + 1) % 2)
        if has_next:
            issue_gather(j + 1, (k + 1) % RD, (k + 1) % 2)
        if has_idx3:
            issue_idx(j + 3, (k + 3) % RD)
        wait_gather(j, k, k % 2)
        scale(bufs[k % 2], j)
        wait_idxD(j, k)
        issue_scatter(j, k, k % 2)

    for k in range(3):
        issue_idx(k, k)
    wait_idxS(0, 0)
    issue_gather(0, 0, 0)
    for j in range(4):
        step(j, j % RD, j == 0, True, True)

    def body(i, _):
        j0 = i * 4
        for k in range(4):
            step(j0 + k, k, False, True, True)
        return 0

    lax.fori_loop(1, (NCH - 4) // 4, body, 0)

    step(NCH - 4, 0, False, True, True)
    step(NCH - 3, 1, False, True, False)
    step(NCH - 2, 2, False, True, False)
    step(NCH - 1, 3, False, False, False)
    wait_scatter(NCH - 1, (NCH - 1) % RD, (NCH - 1) % 2)

    plsc.subcore_barrier()
    for off in range(0, RPT, CB):
        nr = min(CB, RPT - off)
        pltpu.sync_copy(acc_sh.at[pl.ds(row0 + off, nr)],
                        out_hbm.at[c, pl.ds(row0 + off, nr)])


def _segsum_partials(h, srcc, dstc, ew_p):
    f = pl.kernel(
        _segsum_sc_body,
        out_type=jax.ShapeDtypeStruct((NC, NR, D), jnp.float32),
        mesh=_get_sc_mesh(),
        scratch_types=[
            pltpu.VMEM((RD, CB), jnp.int32),
            pltpu.VMEM((RD, CB), jnp.int32),
            pltpu.VMEM((EPT,), jnp.float32),
            pltpu.VMEM((CB, D), jnp.float32),
            pltpu.VMEM((CB, D), jnp.float32),
            pltpu.VMEM_SHARED((NR, D), jnp.float32),
            pltpu.SemaphoreType.DMA,
            pltpu.SemaphoreType.DMA,
            pltpu.SemaphoreType.DMA,
            pltpu.SemaphoreType.DMA,
            pltpu.SemaphoreType.DMA,
            pltpu.SemaphoreType.DMA,
            pltpu.SemaphoreType.DMA,
            pltpu.SemaphoreType.DMA,
            pltpu.SemaphoreType.DMA,
            pltpu.SemaphoreType.DMA,
            pltpu.SemaphoreType.DMA,
            pltpu.SemaphoreType.DMA,
        ],
    )
    return f(h, srcc, dstc, ew_p)


def _segsum(h, srcc, dstc, ew_p):
    out = _segsum_partials(h, srcc, dstc, ew_p)
    return out[0, :N], out[1, :N]


def kernel(x, edge_attr, edge_index, W0, lin0W, W1, lin1W,
           m0_l1W, m0_l1b, m0_l2W, m0_l2b, m0_m1W, m0_m1b, m0_m2W, m0_m2b,
           m1_l1W, m1_l1b, m1_l2W, m1_l2b, m1_m1W, m1_m1b, m1_m2W, m1_m2b,
           p1W, p1b, p2W, p2b):
    pad = E_PAD - edge_attr.shape[0]
    srcc = jnp.pad(edge_index[0], (0, pad)).reshape(E_PAD // CB, CB)
    dstc = jnp.pad(edge_index[1], (0, pad)).reshape(E_PAD // CB, CB)
    ew = jnp.pad(edge_attr, (0, pad))

    zb = jnp.zeros((D,), jnp.float32)

    h1 = _mm(x, m0_l1W, m0_l1b)
    agA, agB = _segsum(h1, srcc, dstc, ew)
    xl2 = _mm(x, m0_l2W, m0_l2b)
    mask0 = _mask_mlp(agA, agB, xl2, m0_m1W, m0_m1b, m0_m2W, m0_m2b)

    h2 = _masked_mm(x, mask0, W0, zb)
    agA, agB = _segsum(h2, srcc, dstc, ew)
    x1 = _combine(agA, agB, x, lin0W, mask0)

    h1 = _masked_mm(x1, mask0, m1_l1W, m1_l1b)
    agA, agB = _segsum(h1, srcc, dstc, ew)
    xl2 = _masked_mm(x1, mask0, m1_l2W, m1_l2b)
    mask1 = _mask_mlp(agA, agB, xl2, m1_m1W, m1_m1b, m1_m2W, m1_m2b)

    h2 = _masked_mm(x1, mask1, W1, zb)
    agA, agB = _segsum(h2, srcc, dstc, ew)
    x2 = _combine(agA, agB, x1, lin1W, mask1)

    return _post(x2, p1W, p1b, p2W, p2b)
